# Initial kernel scaffold; baseline (speedup 1.0000x reference)
import jax
import jax.numpy as jnp
from jax.experimental import pallas as pl


def kernel(x_scalar, x_spherical, rbf, fcut, rsh, edge_index, W1, b1, W2, b2, Wr, br, ln_g, ln_b, o3_g, o3_b, o3_w1, o3_w2):
    raise NotImplementedError("write your pallas kernel here")


# speedup vs baseline: 1.4293x; 1.4293x over previous
import jax
import jax.numpy as jnp
from jax import lax
from jax.experimental import pallas as pl
from jax.experimental.pallas import tpu as pltpu
from jax.experimental.pallas import tpu_sc as plsc

N = 10000
E = 160000
ND = 128
NBASIS = 20
MUL0, MUL1, MUL2 = 128, 64, 32
DSPH = 480
DSPH_P = 512
NIR = 224
HID = 576
EPS = 1e-5

NCORES, NSUB = 2, 16
NW = NCORES * NSUB
EPW = E // NW
EPT = E // NSUB
KA = 40
KB = 40
CW = 128
RPT = 624
LANES = 16

RB = 2000
EB = 2000


def _erep(ncols):
    col = lax.broadcasted_iota(jnp.int32, (NIR, ncols), 1)
    row = lax.broadcasted_iota(jnp.int32, (NIR, ncols), 0)
    r = jnp.where(col < MUL0, col,
        jnp.where(col < MUL0 + 3 * MUL1, MUL0 + (col - MUL0) // 3,
                  MUL0 + MUL1 + (col - MUL0 - 3 * MUL1) // 5))
    return (r == row).astype(jnp.float32)


def _node_body(xs_ref, xsph_ref, W1_ref, b1_ref, W2_ref, b2_ref,
               lng_ref, lnb_ref, o3g_ref, o3b_ref, w1e_ref, w2e_ref,
               A_ref, B_ref, soms_ref):
    xs = xs_ref[...]
    mu = jnp.mean(xs, axis=-1, keepdims=True)
    var = jnp.mean((xs - mu) * (xs - mu), axis=-1, keepdims=True)
    sin = (xs - mu) * lax.rsqrt(var + EPS) * lng_ref[...] + lnb_ref[...]
    h = jnp.dot(sin, W1_ref[...].T, preferred_element_type=jnp.float32) + b1_ref[...]
    h = h * jax.nn.sigmoid(h)
    so = jnp.dot(h, W2_ref[...].T, preferred_element_type=jnp.float32) + b2_ref[...]

    sph = xsph_ref[...]
    s = sph[:, :MUL0]
    mu2 = jnp.mean(s, axis=-1, keepdims=True)
    var2 = jnp.mean((s - mu2) * (s - mu2), axis=-1, keepdims=True)
    sn = (s - mu2) * lax.rsqrt(var2 + EPS) * o3g_ref[...] + o3b_ref[...]
    v1 = sph[:, MUL0:MUL0 + 3 * MUL1]
    inv1 = lax.rsqrt(jnp.sum(v1 * v1, axis=-1, keepdims=True) / MUL1 + EPS)
    v1n = v1 * inv1 * w1e_ref[...]
    v2 = sph[:, MUL0 + 3 * MUL1:]
    inv2 = lax.rsqrt(jnp.sum(v2 * v2, axis=-1, keepdims=True) / MUL2 + EPS)
    v2n = v2 * inv2 * w2e_ref[...]
    pad = jnp.zeros((sph.shape[0], DSPH_P - DSPH), jnp.float32)
    sph_in = jnp.concatenate([sn, v1n, v2n, pad], axis=-1)

    erep = _erep(DSPH_P)
    A_ref[...] = sph_in * jnp.dot(so[:, :NIR], erep, preferred_element_type=jnp.float32)
    B_ref[...] = jnp.dot(so[:, NIR:2 * NIR], erep, preferred_element_type=jnp.float32)
    soms_ref[...] = so[:, 2 * NIR:]


def _edge_body(rbf_ref, fcut_ref, rsh_ref, Wr_ref, br_ref,
               FWS_ref, RBE_ref, FWMS_ref):
    fw = (jnp.dot(rbf_ref[...], Wr_ref[...].T, preferred_element_type=jnp.float32)
          + br_ref[...]) * fcut_ref[...]
    erep = _erep(DSPH)
    FWS_ref[...] = jnp.dot(fw[:, :NIR], erep, preferred_element_type=jnp.float32)
    RBE_ref[...] = rsh_ref[...] * jnp.dot(fw[:, NIR:2 * NIR], erep,
                                          preferred_element_type=jnp.float32)
    FWMS_ref[...] = fw[:, 2 * NIR:]


def _gather_body(A_h, B_h, SOMS_h, FWS_h, RBE_h, FWMS_h, src_h,
                 MS_h, M0_h, M1_h, M2_h, M3_h,
                 idx_v, a_v, b_v, soms_v, fws_v, rbe_v, fwms_v,
                 oms_v, os0_v, os1_v, os2_v, os3_v):
    wid = lax.axis_index("s") * NCORES + lax.axis_index("c")
    base0 = wid * EPW
    zeros = jnp.zeros((LANES,), jnp.float32)

    def chunk(j, carry):
        base = base0 + j * KA
        pltpu.sync_copy(src_h.at[pl.ds(base, KA)], idx_v)
        pltpu.sync_copy(A_h.at[idx_v], a_v)
        pltpu.sync_copy(B_h.at[idx_v], b_v)
        pltpu.sync_copy(SOMS_h.at[idx_v], soms_v)
        pltpu.sync_copy(FWS_h.at[pl.ds(base, KA)], fws_v)
        pltpu.sync_copy(RBE_h.at[pl.ds(base, KA)], rbe_v)
        pltpu.sync_copy(FWMS_h.at[pl.ds(base, KA)], fwms_v)

        obufs = (os0_v, os1_v, os2_v, os3_v)

        def row(r, rcarry):
            for c in range(DSPH // LANES):
                t = c * LANES
                val = (a_v[r, pl.ds(t, LANES)] * fws_v[r, pl.ds(t, LANES)]
                       + b_v[r, pl.ds(t, LANES)] * rbe_v[r, pl.ds(t, LANES)])
                obufs[c // 8][r, pl.ds((c % 8) * LANES, LANES)] = val
            os3_v[r, pl.ds(96, LANES)] = zeros
            os3_v[r, pl.ds(112, LANES)] = zeros
            for c in range(ND // LANES):
                t = c * LANES
                oms_v[r, pl.ds(t, LANES)] = (soms_v[r, pl.ds(t, LANES)]
                                             * fwms_v[r, pl.ds(t, LANES)])
            return rcarry

        lax.fori_loop(0, KA, row, 0)

        pltpu.sync_copy(oms_v, MS_h.at[pl.ds(base, KA)])
        pltpu.sync_copy(os0_v, M0_h.at[pl.ds(base, KA)])
        pltpu.sync_copy(os1_v, M1_h.at[pl.ds(base, KA)])
        pltpu.sync_copy(os2_v, M2_h.at[pl.ds(base, KA)])
        pltpu.sync_copy(os3_v, M3_h.at[pl.ds(base, KA)])
        return carry

    lax.fori_loop(0, EPW // KA, chunk, 0)


def _scatter_body(MS_h, M0_h, M1_h, M2_h, M3_h, dst_h,
                  XS_h, X0_h, X1_h, X2_h, X3_h,
                  OMS_h, O0_h, O1_h, O2_h, O3_h,
                  acc_sh, idx_v, rows_v):
    core = lax.axis_index("c")
    sub = lax.axis_index("s")

    def do_chunk(msg_h, x_h, out_h):
        pltpu.sync_copy(x_h.at[pl.ds(sub * RPT, RPT)],
                        acc_sh.at[pl.ds(sub * RPT, RPT)])

        @pl.when(sub == NSUB - 1)
        def _():
            pltpu.sync_copy(x_h.at[pl.ds(NSUB * RPT, N - NSUB * RPT)],
                            acc_sh.at[pl.ds(NSUB * RPT, N - NSUB * RPT)])

        plsc.subcore_barrier()
        base0 = sub * EPT

        def step(j, carry):
            base = base0 + j * KB
            pltpu.sync_copy(dst_h.at[pl.ds(base, KB)], idx_v)
            pltpu.sync_copy(msg_h.at[pl.ds(base, KB)], rows_v)
            pltpu.sync_copy(rows_v, acc_sh.at[idx_v], add=True)
            return carry

        lax.fori_loop(0, EPT // KB, step, 0)
        plsc.subcore_barrier()
        pltpu.sync_copy(acc_sh.at[pl.ds(sub * RPT, RPT)],
                        out_h.at[pl.ds(sub * RPT, RPT)])

        @pl.when(sub == NSUB - 1)
        def _():
            pltpu.sync_copy(acc_sh.at[pl.ds(NSUB * RPT, N - NSUB * RPT)],
                            out_h.at[pl.ds(NSUB * RPT, N - NSUB * RPT)])

        plsc.subcore_barrier()

    @pl.when(core == 0)
    def _():
        do_chunk(MS_h, XS_h, OMS_h)
        do_chunk(M0_h, X0_h, O0_h)
        do_chunk(M1_h, X1_h, O1_h)

    @pl.when(core == 1)
    def _():
        do_chunk(M2_h, X2_h, O2_h)
        do_chunk(M3_h, X3_h, O3_h)


def _f32(*shape):
    return jax.ShapeDtypeStruct(shape, jnp.float32)


def kernel(x_scalar, x_spherical, rbf, fcut, rsh, edge_index,
           W1, b1, W2, b2, Wr, br, ln_g, ln_b, o3_g, o3_b, o3_w1, o3_w2):
    src = edge_index[1]
    dst = edge_index[0]
    w1e = jnp.repeat(o3_w1, 3).reshape(1, -1)
    w2e = jnp.repeat(o3_w2, 5).reshape(1, -1)

    full = lambda s: pl.BlockSpec(s, lambda i: (0, 0))
    A, B, SOMS = pl.pallas_call(
        _node_body,
        grid=(N // RB,),
        in_specs=[
            pl.BlockSpec((RB, ND), lambda i: (i, 0)),
            pl.BlockSpec((RB, DSPH), lambda i: (i, 0)),
            full((ND, ND)), full((1, ND)), full((HID, ND)), full((1, HID)),
            full((1, ND)), full((1, ND)), full((1, MUL0)), full((1, MUL0)),
            full((1, 3 * MUL1)), full((1, 5 * MUL2)),
        ],
        out_specs=[
            pl.BlockSpec((RB, DSPH_P), lambda i: (i, 0)),
            pl.BlockSpec((RB, DSPH_P), lambda i: (i, 0)),
            pl.BlockSpec((RB, ND), lambda i: (i, 0)),
        ],
        out_shape=[_f32(N, DSPH_P), _f32(N, DSPH_P), _f32(N, ND)],
    )(x_scalar, x_spherical, W1, b1.reshape(1, -1), W2, b2.reshape(1, -1),
      ln_g.reshape(1, -1), ln_b.reshape(1, -1), o3_g.reshape(1, -1),
      o3_b.reshape(1, -1), w1e, w2e)

    FWS, RBE, FWMS = pl.pallas_call(
        _edge_body,
        grid=(E // EB,),
        in_specs=[
            pl.BlockSpec((EB, NBASIS), lambda i: (i, 0)),
            pl.BlockSpec((EB, 1), lambda i: (i, 0)),
            pl.BlockSpec((EB, DSPH), lambda i: (i, 0)),
            full((HID, NBASIS)), full((1, HID)),
        ],
        out_specs=[
            pl.BlockSpec((EB, DSPH), lambda i: (i, 0)),
            pl.BlockSpec((EB, DSPH), lambda i: (i, 0)),
            pl.BlockSpec((EB, ND), lambda i: (i, 0)),
        ],
        out_shape=[_f32(E, DSPH), _f32(E, DSPH), _f32(E, ND)],
    )(rbf, fcut, rsh, Wr, br.reshape(1, -1))

    mesh = plsc.VectorSubcoreMesh(core_axis_name="c", subcore_axis_name="s")

    MS, M0, M1, M2, M3 = pl.kernel(
        _gather_body,
        out_type=[_f32(E, CW)] * 5,
        mesh=mesh,
        scratch_types=[
            pltpu.VMEM((KA,), jnp.int32),
            pltpu.VMEM((KA, DSPH_P), jnp.float32),
            pltpu.VMEM((KA, DSPH_P), jnp.float32),
            pltpu.VMEM((KA, ND), jnp.float32),
            pltpu.VMEM((KA, DSPH), jnp.float32),
            pltpu.VMEM((KA, DSPH), jnp.float32),
            pltpu.VMEM((KA, ND), jnp.float32),
            pltpu.VMEM((KA, CW), jnp.float32),
            pltpu.VMEM((KA, CW), jnp.float32),
            pltpu.VMEM((KA, CW), jnp.float32),
            pltpu.VMEM((KA, CW), jnp.float32),
            pltpu.VMEM((KA, CW), jnp.float32),
        ],
    )(A, B, SOMS, FWS, RBE, FWMS, src)

    x0 = x_spherical[:, :CW]
    x1 = x_spherical[:, CW:2 * CW]
    x2 = x_spherical[:, 2 * CW:3 * CW]
    x3 = jnp.pad(x_spherical[:, 3 * CW:], ((0, 0), (0, 4 * CW - DSPH)))

    oms, o0, o1, o2, o3 = pl.kernel(
        _scatter_body,
        out_type=[_f32(N, CW)] * 5,
        mesh=mesh,
        scratch_types=[
            pltpu.VMEM_SHARED((N, CW), jnp.float32),
            pltpu.VMEM((KB,), jnp.int32),
            pltpu.VMEM((KB, CW), jnp.float32),
        ],
    )(MS, M0, M1, M2, M3, dst, x_scalar, x0, x1, x2, x3)

    new_scalar = oms
    new_spherical = jnp.concatenate([o0, o1, o2, o3[:, :DSPH - 3 * CW]], axis=1)
    return (new_scalar, new_spherical)


# speedup vs baseline: 1.8685x; 1.3073x over previous
import jax
import jax.numpy as jnp
from jax import lax
from jax.experimental import pallas as pl
from jax.experimental.pallas import tpu as pltpu
from jax.experimental.pallas import tpu_sc as plsc

N = 10000
E = 160000
ND = 128
NBASIS = 20
MUL0, MUL1, MUL2 = 128, 64, 32
DSPH = 480
DSPH_P = 512
DABM = DSPH_P + ND
NIR = 224
HID = 576
EPS = 1e-5

NCORES, NSUB = 2, 16
NW = NCORES * NSUB
EPW = E // NW
EPT = E // NSUB
KA = 40
KB = 40
CW = 128
RPT = 624
LANES = 16

RB = 2000
EB = 2000


def _erep(ncols):
    col = lax.broadcasted_iota(jnp.int32, (NIR, ncols), 1)
    row = lax.broadcasted_iota(jnp.int32, (NIR, ncols), 0)
    r = jnp.where(col < MUL0, col,
        jnp.where(col < MUL0 + 3 * MUL1, MUL0 + (col - MUL0) // 3,
                  MUL0 + MUL1 + (col - MUL0 - 3 * MUL1) // 5))
    return (r == row).astype(jnp.float32)


def _node_body(xs_ref, xsph_ref, W1_ref, b1_ref, W2_ref, b2_ref,
               lng_ref, lnb_ref, o3g_ref, o3b_ref, w1e_ref, w2e_ref,
               ABM_ref, B_ref):
    xs = xs_ref[...]
    mu = jnp.mean(xs, axis=-1, keepdims=True)
    var = jnp.mean((xs - mu) * (xs - mu), axis=-1, keepdims=True)
    sin = (xs - mu) * lax.rsqrt(var + EPS) * lng_ref[...] + lnb_ref[...]
    h = jnp.dot(sin, W1_ref[...].T, preferred_element_type=jnp.float32) + b1_ref[...]
    h = h * jax.nn.sigmoid(h)
    so = jnp.dot(h, W2_ref[...].T, preferred_element_type=jnp.float32) + b2_ref[...]

    sph = xsph_ref[...]
    s = sph[:, :MUL0]
    mu2 = jnp.mean(s, axis=-1, keepdims=True)
    var2 = jnp.mean((s - mu2) * (s - mu2), axis=-1, keepdims=True)
    sn = (s - mu2) * lax.rsqrt(var2 + EPS) * o3g_ref[...] + o3b_ref[...]
    v1 = sph[:, MUL0:MUL0 + 3 * MUL1]
    inv1 = lax.rsqrt(jnp.sum(v1 * v1, axis=-1, keepdims=True) / MUL1 + EPS)
    v1n = v1 * inv1 * w1e_ref[...]
    v2 = sph[:, MUL0 + 3 * MUL1:]
    inv2 = lax.rsqrt(jnp.sum(v2 * v2, axis=-1, keepdims=True) / MUL2 + EPS)
    v2n = v2 * inv2 * w2e_ref[...]
    pad = jnp.zeros((sph.shape[0], DSPH_P - DSPH), jnp.float32)
    sph_in = jnp.concatenate([sn, v1n, v2n, pad], axis=-1)

    erep = _erep(DSPH_P)
    a = sph_in * jnp.dot(so[:, :NIR], erep, preferred_element_type=jnp.float32)
    ABM_ref[...] = jnp.concatenate([a, so[:, 2 * NIR:]], axis=-1)
    B_ref[...] = jnp.dot(so[:, NIR:2 * NIR], erep, preferred_element_type=jnp.float32)


def _edge_body(rbf_ref, fcut_ref, rsh_ref, Wr_ref, br_ref,
               FWS_ref, RBE_ref, FWMS_ref):
    fw = (jnp.dot(rbf_ref[...], Wr_ref[...].T, preferred_element_type=jnp.float32)
          + br_ref[...]) * fcut_ref[...]
    erep = _erep(DSPH_P)
    FWS_ref[...] = jnp.dot(fw[:, :NIR], erep, preferred_element_type=jnp.float32)
    pad = jnp.zeros((fw.shape[0], DSPH_P - DSPH), jnp.float32)
    rshp = jnp.concatenate([rsh_ref[...], pad], axis=-1)
    RBE_ref[...] = rshp * jnp.dot(fw[:, NIR:2 * NIR], erep,
                                  preferred_element_type=jnp.float32)
    FWMS_ref[...] = fw[:, 2 * NIR:]


def _gather_body(ABM_h, B_h, FWS_h, RBE_h, FWMS_h, src_h,
                 MS_h, M0_h, M1_h, M2_h, M3_h,
                 idx_all, ab_v, b_v, fws_v, rbe_v, fwms_v,
                 oms_v, os0_v, os1_v, os2_v, os3_v, sem_in, sem_out):
    wid = lax.axis_index("s") * NCORES + lax.axis_index("c")
    base0 = wid * EPW
    zeros = jnp.zeros((LANES,), jnp.float32)
    pltpu.sync_copy(src_h.at[pl.ds(base0, EPW)], idx_all)

    obufs = (os0_v, os1_v, os2_v, os3_v)
    opairs = ((oms_v, MS_h), (os0_v, M0_h), (os1_v, M1_h),
              (os2_v, M2_h), (os3_v, M3_h))

    def chunk(j, carry):
        base = base0 + j * KA
        idx = idx_all.at[pl.ds(j * KA, KA)]
        d_ab = pltpu.async_copy(ABM_h.at[idx], ab_v, sem_in)
        d_b = pltpu.async_copy(B_h.at[idx], b_v, sem_in)
        d_f1 = pltpu.async_copy(FWS_h.at[pl.ds(base, KA)], fws_v, sem_in)
        d_f2 = pltpu.async_copy(RBE_h.at[pl.ds(base, KA)], rbe_v, sem_in)
        d_f3 = pltpu.async_copy(FWMS_h.at[pl.ds(base, KA)], fwms_v, sem_in)

        @pl.when(j > 0)
        def _():
            for buf, hbm in opairs:
                pltpu.make_async_copy(buf, hbm.at[pl.ds(base, KA)],
                                      sem_out).wait()

        d_ab.wait()
        d_b.wait()
        d_f1.wait()
        d_f2.wait()
        d_f3.wait()

        def row(r, rcarry):
            for c in range(DSPH // LANES):
                t = c * LANES
                val = (ab_v[r, pl.ds(t, LANES)] * fws_v[r, pl.ds(t, LANES)]
                       + b_v[r, pl.ds(t, LANES)] * rbe_v[r, pl.ds(t, LANES)])
                obufs[c // 8][r, pl.ds((c % 8) * LANES, LANES)] = val
            os3_v[r, pl.ds(96, LANES)] = zeros
            os3_v[r, pl.ds(112, LANES)] = zeros
            for c in range(ND // LANES):
                t = c * LANES
                oms_v[r, pl.ds(t, LANES)] = (
                    ab_v[r, pl.ds(DSPH_P + t, LANES)]
                    * fwms_v[r, pl.ds(t, LANES)])
            return rcarry

        lax.fori_loop(0, KA, row, 0)

        for buf, hbm in opairs:
            pltpu.async_copy(buf, hbm.at[pl.ds(base, KA)], sem_out)
        return carry

    lax.fori_loop(0, EPW // KA, chunk, 0)
    last = base0 + EPW - KA
    for buf, hbm in opairs:
        pltpu.make_async_copy(buf, hbm.at[pl.ds(last, KA)], sem_out).wait()


def _scatter_body(MS_h, M0_h, M1_h, M2_h, M3_h, dst_h,
                  XS_h, X0_h, X1_h, X2_h, X3_h,
                  OMS_h, O0_h, O1_h, O2_h, O3_h,
                  acc_sh, idxa_v, idxb_v, rowsa_v, rowsb_v,
                  sema, semb, sems):
    core = lax.axis_index("c")
    sub = lax.axis_index("s")

    def do_chunk(msg_h, x_h, out_h):
        pltpu.sync_copy(x_h.at[pl.ds(sub * RPT, RPT)],
                        acc_sh.at[pl.ds(sub * RPT, RPT)])

        @pl.when(sub == NSUB - 1)
        def _():
            pltpu.sync_copy(x_h.at[pl.ds(NSUB * RPT, N - NSUB * RPT)],
                            acc_sh.at[pl.ds(NSUB * RPT, N - NSUB * RPT)])

        plsc.subcore_barrier()
        base0 = sub * EPT
        npairs = EPT // (2 * KB)

        def pair(p, carry):
            basea = base0 + p * 2 * KB
            baseb = basea + KB

            @pl.when(p > 0)
            def _():
                pltpu.make_async_copy(rowsa_v, acc_sh.at[idxa_v], sems).wait()
                pltpu.make_async_copy(rowsb_v, acc_sh.at[idxb_v], sems).wait()

            da1 = pltpu.async_copy(dst_h.at[pl.ds(basea, KB)], idxa_v, sema)
            da2 = pltpu.async_copy(msg_h.at[pl.ds(basea, KB)], rowsa_v, sema)
            db1 = pltpu.async_copy(dst_h.at[pl.ds(baseb, KB)], idxb_v, semb)
            db2 = pltpu.async_copy(msg_h.at[pl.ds(baseb, KB)], rowsb_v, semb)
            da1.wait()
            da2.wait()
            pltpu.async_copy(rowsa_v, acc_sh.at[idxa_v], sems, add=True)
            db1.wait()
            db2.wait()
            pltpu.async_copy(rowsb_v, acc_sh.at[idxb_v], sems, add=True)
            return carry

        lax.fori_loop(0, npairs, pair, 0)
        pltpu.make_async_copy(rowsa_v, acc_sh.at[idxa_v], sems).wait()
        pltpu.make_async_copy(rowsb_v, acc_sh.at[idxb_v], sems).wait()
        plsc.subcore_barrier()
        pltpu.sync_copy(acc_sh.at[pl.ds(sub * RPT, RPT)],
                        out_h.at[pl.ds(sub * RPT, RPT)])

        @pl.when(sub == NSUB - 1)
        def _():
            pltpu.sync_copy(acc_sh.at[pl.ds(NSUB * RPT, N - NSUB * RPT)],
                            out_h.at[pl.ds(NSUB * RPT, N - NSUB * RPT)])

        plsc.subcore_barrier()

    @pl.when(core == 0)
    def _():
        do_chunk(MS_h, XS_h, OMS_h)
        do_chunk(M0_h, X0_h, O0_h)
        do_chunk(M1_h, X1_h, O1_h)

    @pl.when(core == 1)
    def _():
        do_chunk(M2_h, X2_h, O2_h)
        do_chunk(M3_h, X3_h, O3_h)


def _f32(*shape):
    return jax.ShapeDtypeStruct(shape, jnp.float32)


def kernel(x_scalar, x_spherical, rbf, fcut, rsh, edge_index,
           W1, b1, W2, b2, Wr, br, ln_g, ln_b, o3_g, o3_b, o3_w1, o3_w2):
    src = edge_index[1]
    dst = edge_index[0]
    w1e = jnp.repeat(o3_w1, 3).reshape(1, -1)
    w2e = jnp.repeat(o3_w2, 5).reshape(1, -1)

    full = lambda s: pl.BlockSpec(s, lambda i: (0, 0))
    ABM, B = pl.pallas_call(
        _node_body,
        grid=(N // RB,),
        in_specs=[
            pl.BlockSpec((RB, ND), lambda i: (i, 0)),
            pl.BlockSpec((RB, DSPH), lambda i: (i, 0)),
            full((ND, ND)), full((1, ND)), full((HID, ND)), full((1, HID)),
            full((1, ND)), full((1, ND)), full((1, MUL0)), full((1, MUL0)),
            full((1, 3 * MUL1)), full((1, 5 * MUL2)),
        ],
        out_specs=[
            pl.BlockSpec((RB, DABM), lambda i: (i, 0)),
            pl.BlockSpec((RB, DSPH_P), lambda i: (i, 0)),
        ],
        out_shape=[_f32(N, DABM), _f32(N, DSPH_P)],
    )(x_scalar, x_spherical, W1, b1.reshape(1, -1), W2, b2.reshape(1, -1),
      ln_g.reshape(1, -1), ln_b.reshape(1, -1), o3_g.reshape(1, -1),
      o3_b.reshape(1, -1), w1e, w2e)

    FWS, RBE, FWMS = pl.pallas_call(
        _edge_body,
        grid=(E // EB,),
        in_specs=[
            pl.BlockSpec((EB, NBASIS), lambda i: (i, 0)),
            pl.BlockSpec((EB, 1), lambda i: (i, 0)),
            pl.BlockSpec((EB, DSPH), lambda i: (i, 0)),
            full((HID, NBASIS)), full((1, HID)),
        ],
        out_specs=[
            pl.BlockSpec((EB, DSPH_P), lambda i: (i, 0)),
            pl.BlockSpec((EB, DSPH_P), lambda i: (i, 0)),
            pl.BlockSpec((EB, ND), lambda i: (i, 0)),
        ],
        out_shape=[_f32(E, DSPH_P), _f32(E, DSPH_P), _f32(E, ND)],
    )(rbf, fcut, rsh, Wr, br.reshape(1, -1))

    mesh = plsc.VectorSubcoreMesh(core_axis_name="c", subcore_axis_name="s")

    MS, M0, M1, M2, M3 = pl.kernel(
        _gather_body,
        out_type=[_f32(E, CW)] * 5,
        mesh=mesh,
        scratch_types=[
            pltpu.VMEM((EPW,), jnp.int32),
            pltpu.VMEM((KA, DABM), jnp.float32),
            pltpu.VMEM((KA, DSPH_P), jnp.float32),
            pltpu.VMEM((KA, DSPH_P), jnp.float32),
            pltpu.VMEM((KA, DSPH_P), jnp.float32),
            pltpu.VMEM((KA, ND), jnp.float32),
            pltpu.VMEM((KA, CW), jnp.float32),
            pltpu.VMEM((KA, CW), jnp.float32),
            pltpu.VMEM((KA, CW), jnp.float32),
            pltpu.VMEM((KA, CW), jnp.float32),
            pltpu.VMEM((KA, CW), jnp.float32),
            pltpu.SemaphoreType.DMA,
            pltpu.SemaphoreType.DMA,
        ],
    )(ABM, B, FWS, RBE, FWMS, src)

    x0 = x_spherical[:, :CW]
    x1 = x_spherical[:, CW:2 * CW]
    x2 = x_spherical[:, 2 * CW:3 * CW]
    x3 = jnp.pad(x_spherical[:, 3 * CW:], ((0, 0), (0, 4 * CW - DSPH)))

    oms, o0, o1, o2, o3 = pl.kernel(
        _scatter_body,
        out_type=[_f32(N, CW)] * 5,
        mesh=mesh,
        scratch_types=[
            pltpu.VMEM_SHARED((N, CW), jnp.float32),
            pltpu.VMEM((KB,), jnp.int32),
            pltpu.VMEM((KB,), jnp.int32),
            pltpu.VMEM((KB, CW), jnp.float32),
            pltpu.VMEM((KB, CW), jnp.float32),
            pltpu.SemaphoreType.DMA,
            pltpu.SemaphoreType.DMA,
            pltpu.SemaphoreType.DMA,
        ],
    )(MS, M0, M1, M2, M3, dst, x_scalar, x0, x1, x2, x3)

    new_scalar = oms
    new_spherical = jnp.concatenate([o0, o1, o2, o3[:, :DSPH - 3 * CW]], axis=1)
    return (new_scalar, new_spherical)


# speedup vs baseline: 2.3060x; 1.2342x over previous
import jax
import jax.numpy as jnp
from jax import lax
from jax.experimental import pallas as pl
from jax.experimental.pallas import tpu as pltpu
from jax.experimental.pallas import tpu_sc as plsc

N = 10000
E = 160000
ND = 128
NBASIS = 20
MUL0, MUL1, MUL2 = 128, 64, 32
DSPH = 480
DSPH_P = 512
DABM = DSPH_P + ND
NIR = 224
HID = 576
EPS = 1e-5

NCORES, NSUB = 2, 16
NW = NCORES * NSUB
EPW = E // NW
EPT = E // NSUB
KA = 40
KB = 40
CW = 128
RPT = 624
LANES = 16

RB = 2000
EB = 2000


def _erep(ncols):
    col = lax.broadcasted_iota(jnp.int32, (NIR, ncols), 1)
    row = lax.broadcasted_iota(jnp.int32, (NIR, ncols), 0)
    r = jnp.where(col < MUL0, col,
        jnp.where(col < MUL0 + 3 * MUL1, MUL0 + (col - MUL0) // 3,
                  MUL0 + MUL1 + (col - MUL0 - 3 * MUL1) // 5))
    return (r == row).astype(jnp.float32)


def _node_body(xs_ref, xsph_ref, W1_ref, b1_ref, W2_ref, b2_ref,
               lng_ref, lnb_ref, o3g_ref, o3b_ref, w1e_ref, w2e_ref,
               ABM_ref, B_ref):
    xs = xs_ref[...]
    mu = jnp.mean(xs, axis=-1, keepdims=True)
    var = jnp.mean((xs - mu) * (xs - mu), axis=-1, keepdims=True)
    sin = (xs - mu) * lax.rsqrt(var + EPS) * lng_ref[...] + lnb_ref[...]
    h = jnp.dot(sin, W1_ref[...].T, preferred_element_type=jnp.float32) + b1_ref[...]
    h = h * jax.nn.sigmoid(h)
    so = jnp.dot(h, W2_ref[...].T, preferred_element_type=jnp.float32) + b2_ref[...]

    sph = xsph_ref[...]
    s = sph[:, :MUL0]
    mu2 = jnp.mean(s, axis=-1, keepdims=True)
    var2 = jnp.mean((s - mu2) * (s - mu2), axis=-1, keepdims=True)
    sn = (s - mu2) * lax.rsqrt(var2 + EPS) * o3g_ref[...] + o3b_ref[...]
    v1 = sph[:, MUL0:MUL0 + 3 * MUL1]
    inv1 = lax.rsqrt(jnp.sum(v1 * v1, axis=-1, keepdims=True) / MUL1 + EPS)
    v1n = v1 * inv1 * w1e_ref[...]
    v2 = sph[:, MUL0 + 3 * MUL1:]
    inv2 = lax.rsqrt(jnp.sum(v2 * v2, axis=-1, keepdims=True) / MUL2 + EPS)
    v2n = v2 * inv2 * w2e_ref[...]
    pad = jnp.zeros((sph.shape[0], DSPH_P - DSPH), jnp.float32)
    sph_in = jnp.concatenate([sn, v1n, v2n, pad], axis=-1)

    erep = _erep(DSPH_P)
    a = sph_in * jnp.dot(so[:, :NIR], erep, preferred_element_type=jnp.float32)
    ABM_ref[...] = jnp.concatenate([a, so[:, 2 * NIR:]], axis=-1)
    B_ref[...] = jnp.dot(so[:, NIR:2 * NIR], erep, preferred_element_type=jnp.float32)


def _edge_body(rbf_ref, fcut_ref, rsh_ref, Wr_ref, br_ref, FWX_ref):
    fw = (jnp.dot(rbf_ref[...], Wr_ref[...].T, preferred_element_type=jnp.float32)
          + br_ref[...]) * fcut_ref[...]
    erep = _erep(DSPH_P)
    fws = jnp.dot(fw[:, :NIR], erep, preferred_element_type=jnp.float32)
    pad = jnp.zeros((fw.shape[0], DSPH_P - DSPH), jnp.float32)
    rshp = jnp.concatenate([rsh_ref[...], pad], axis=-1)
    rbe = rshp * jnp.dot(fw[:, NIR:2 * NIR], erep,
                         preferred_element_type=jnp.float32)
    FWX_ref[...] = jnp.concatenate([fws, rbe, fw[:, 2 * NIR:]], axis=-1)


def _gather_body(ABM_h, B_h, FWX_h, src_h,
                 MS_h, M0_h, M1_h, M2_h, M3_h,
                 idx_all, ab_v, b_v, fwx_v,
                 oms_v, os0_v, os1_v, os2_v, os3_v, sem_in, sem_out):
    wid = lax.axis_index("s") * NCORES + lax.axis_index("c")
    base0 = wid * EPW
    zeros = jnp.zeros((LANES,), jnp.float32)
    pltpu.sync_copy(src_h.at[pl.ds(base0, EPW)], idx_all)

    obufs = (os0_v, os1_v, os2_v, os3_v)
    opairs = ((oms_v, MS_h), (os0_v, M0_h), (os1_v, M1_h),
              (os2_v, M2_h), (os3_v, M3_h))

    def chunk(j, carry):
        base = base0 + j * KA
        idx = idx_all.at[pl.ds(j * KA, KA)]
        d_ab = pltpu.async_copy(ABM_h.at[idx], ab_v, sem_in)
        d_b = pltpu.async_copy(B_h.at[idx], b_v, sem_in)
        d_fx = pltpu.async_copy(FWX_h.at[pl.ds(base, KA)], fwx_v, sem_in)

        @pl.when(j > 0)
        def _():
            for buf, hbm in opairs:
                pltpu.make_async_copy(buf, hbm.at[pl.ds(base, KA)],
                                      sem_out).wait()

        d_ab.wait()
        d_b.wait()
        d_fx.wait()

        @plsc.parallel_loop(0, KA, unroll=4)
        def row(r):
            for c in range(DSPH // LANES):
                t = c * LANES
                val = (ab_v[r, pl.ds(t, LANES)] * fwx_v[r, pl.ds(t, LANES)]
                       + b_v[r, pl.ds(t, LANES)]
                       * fwx_v[r, pl.ds(DSPH_P + t, LANES)])
                obufs[c // 8][r, pl.ds((c % 8) * LANES, LANES)] = val
            os3_v[r, pl.ds(96, LANES)] = zeros
            os3_v[r, pl.ds(112, LANES)] = zeros
            for c in range(ND // LANES):
                t = c * LANES
                oms_v[r, pl.ds(t, LANES)] = (
                    ab_v[r, pl.ds(DSPH_P + t, LANES)]
                    * fwx_v[r, pl.ds(2 * DSPH_P + t, LANES)])

        for buf, hbm in opairs:
            pltpu.async_copy(buf, hbm.at[pl.ds(base, KA)], sem_out)
        return carry

    lax.fori_loop(0, EPW // KA, chunk, 0)
    last = base0 + EPW - KA
    for buf, hbm in opairs:
        pltpu.make_async_copy(buf, hbm.at[pl.ds(last, KA)], sem_out).wait()


def _scatter_body(MS_h, M0_h, M1_h, M2_h, M3_h, dst_h,
                  XS_h, X0_h, X1_h, X2_h, X3_h,
                  OMS_h, O0_h, O1_h, O2_h, O3_h,
                  acc_sh, idxa_v, idxb_v, rowsa_v, rowsb_v,
                  sema, semb, sems):
    core = lax.axis_index("c")
    sub = lax.axis_index("s")

    def do_chunk(msg_h, x_h, out_h):
        pltpu.sync_copy(x_h.at[pl.ds(sub * RPT, RPT)],
                        acc_sh.at[pl.ds(sub * RPT, RPT)])

        @pl.when(sub == NSUB - 1)
        def _():
            pltpu.sync_copy(x_h.at[pl.ds(NSUB * RPT, N - NSUB * RPT)],
                            acc_sh.at[pl.ds(NSUB * RPT, N - NSUB * RPT)])

        plsc.subcore_barrier()
        base0 = sub * EPT
        npairs = EPT // (2 * KB)

        def pair(p, carry):
            basea = base0 + p * 2 * KB
            baseb = basea + KB

            @pl.when(p > 0)
            def _():
                pltpu.make_async_copy(rowsa_v, acc_sh.at[idxa_v], sems).wait()
                pltpu.make_async_copy(rowsb_v, acc_sh.at[idxb_v], sems).wait()

            da1 = pltpu.async_copy(dst_h.at[pl.ds(basea, KB)], idxa_v, sema)
            da2 = pltpu.async_copy(msg_h.at[pl.ds(basea, KB)], rowsa_v, sema)
            db1 = pltpu.async_copy(dst_h.at[pl.ds(baseb, KB)], idxb_v, semb)
            db2 = pltpu.async_copy(msg_h.at[pl.ds(baseb, KB)], rowsb_v, semb)
            da1.wait()
            da2.wait()
            pltpu.async_copy(rowsa_v, acc_sh.at[idxa_v], sems, add=True)
            db1.wait()
            db2.wait()
            pltpu.async_copy(rowsb_v, acc_sh.at[idxb_v], sems, add=True)
            return carry

        lax.fori_loop(0, npairs, pair, 0)
        pltpu.make_async_copy(rowsa_v, acc_sh.at[idxa_v], sems).wait()
        pltpu.make_async_copy(rowsb_v, acc_sh.at[idxb_v], sems).wait()
        plsc.subcore_barrier()
        pltpu.sync_copy(acc_sh.at[pl.ds(sub * RPT, RPT)],
                        out_h.at[pl.ds(sub * RPT, RPT)])

        @pl.when(sub == NSUB - 1)
        def _():
            pltpu.sync_copy(acc_sh.at[pl.ds(NSUB * RPT, N - NSUB * RPT)],
                            out_h.at[pl.ds(NSUB * RPT, N - NSUB * RPT)])

        plsc.subcore_barrier()

    @pl.when(core == 0)
    def _():
        do_chunk(MS_h, XS_h, OMS_h)
        do_chunk(M0_h, X0_h, O0_h)
        do_chunk(M1_h, X1_h, O1_h)

    @pl.when(core == 1)
    def _():
        do_chunk(M2_h, X2_h, O2_h)
        do_chunk(M3_h, X3_h, O3_h)


def _f32(*shape):
    return jax.ShapeDtypeStruct(shape, jnp.float32)


def kernel(x_scalar, x_spherical, rbf, fcut, rsh, edge_index,
           W1, b1, W2, b2, Wr, br, ln_g, ln_b, o3_g, o3_b, o3_w1, o3_w2):
    src = edge_index[1]
    dst = edge_index[0]
    w1e = jnp.repeat(o3_w1, 3).reshape(1, -1)
    w2e = jnp.repeat(o3_w2, 5).reshape(1, -1)

    full = lambda s: pl.BlockSpec(s, lambda i: (0, 0))
    ABM, B = pl.pallas_call(
        _node_body,
        grid=(N // RB,),
        in_specs=[
            pl.BlockSpec((RB, ND), lambda i: (i, 0)),
            pl.BlockSpec((RB, DSPH), lambda i: (i, 0)),
            full((ND, ND)), full((1, ND)), full((HID, ND)), full((1, HID)),
            full((1, ND)), full((1, ND)), full((1, MUL0)), full((1, MUL0)),
            full((1, 3 * MUL1)), full((1, 5 * MUL2)),
        ],
        out_specs=[
            pl.BlockSpec((RB, DABM), lambda i: (i, 0)),
            pl.BlockSpec((RB, DSPH_P), lambda i: (i, 0)),
        ],
        out_shape=[_f32(N, DABM), _f32(N, DSPH_P)],
    )(x_scalar, x_spherical, W1, b1.reshape(1, -1), W2, b2.reshape(1, -1),
      ln_g.reshape(1, -1), ln_b.reshape(1, -1), o3_g.reshape(1, -1),
      o3_b.reshape(1, -1), w1e, w2e)

    FWX, = pl.pallas_call(
        _edge_body,
        grid=(E // EB,),
        in_specs=[
            pl.BlockSpec((EB, NBASIS), lambda i: (i, 0)),
            pl.BlockSpec((EB, 1), lambda i: (i, 0)),
            pl.BlockSpec((EB, DSPH), lambda i: (i, 0)),
            full((HID, NBASIS)), full((1, HID)),
        ],
        out_specs=[
            pl.BlockSpec((EB, 2 * DSPH_P + ND), lambda i: (i, 0)),
        ],
        out_shape=[_f32(E, 2 * DSPH_P + ND)],
    )(rbf, fcut, rsh, Wr, br.reshape(1, -1))

    mesh = plsc.VectorSubcoreMesh(core_axis_name="c", subcore_axis_name="s")

    MS, M0, M1, M2, M3 = pl.kernel(
        _gather_body,
        out_type=[_f32(E, CW)] * 5,
        mesh=mesh,
        scratch_types=[
            pltpu.VMEM((EPW,), jnp.int32),
            pltpu.VMEM((KA, DABM), jnp.float32),
            pltpu.VMEM((KA, DSPH_P), jnp.float32),
            pltpu.VMEM((KA, 2 * DSPH_P + ND), jnp.float32),
            pltpu.VMEM((KA, CW), jnp.float32),
            pltpu.VMEM((KA, CW), jnp.float32),
            pltpu.VMEM((KA, CW), jnp.float32),
            pltpu.VMEM((KA, CW), jnp.float32),
            pltpu.VMEM((KA, CW), jnp.float32),
            pltpu.SemaphoreType.DMA,
            pltpu.SemaphoreType.DMA,
        ],
    )(ABM, B, FWX, src)

    x0 = x_spherical[:, :CW]
    x1 = x_spherical[:, CW:2 * CW]
    x2 = x_spherical[:, 2 * CW:3 * CW]
    x3 = jnp.pad(x_spherical[:, 3 * CW:], ((0, 0), (0, 4 * CW - DSPH)))

    oms, o0, o1, o2, o3 = pl.kernel(
        _scatter_body,
        out_type=[_f32(N, CW)] * 5,
        mesh=mesh,
        scratch_types=[
            pltpu.VMEM_SHARED((N, CW), jnp.float32),
            pltpu.VMEM((KB,), jnp.int32),
            pltpu.VMEM((KB,), jnp.int32),
            pltpu.VMEM((KB, CW), jnp.float32),
            pltpu.VMEM((KB, CW), jnp.float32),
            pltpu.SemaphoreType.DMA,
            pltpu.SemaphoreType.DMA,
            pltpu.SemaphoreType.DMA,
        ],
    )(MS, M0, M1, M2, M3, dst, x_scalar, x0, x1, x2, x3)

    new_scalar = oms
    new_spherical = jnp.concatenate([o0, o1, o2, o3[:, :DSPH - 3 * CW]], axis=1)
    return (new_scalar, new_spherical)


# speedup vs baseline: 2.3171x; 1.0048x over previous
import jax
import jax.numpy as jnp
from jax import lax
from jax.experimental import pallas as pl
from jax.experimental.pallas import tpu as pltpu
from jax.experimental.pallas import tpu_sc as plsc

N = 10000
E = 160000
ND = 128
NBASIS = 20
MUL0, MUL1, MUL2 = 128, 64, 32
DSPH = 480
DSPH_P = 512
DABM = DSPH_P + ND
NIR = 224
HID = 576
EPS = 1e-5

NCORES, NSUB = 2, 16
NW = NCORES * NSUB
EPW = E // NW
EPT = E // NSUB
KA = 40
KB = 40
CW = 128
RPT = 624
LANES = 16

RB = 2000
EB = 2000


def _erep(ncols):
    col = lax.broadcasted_iota(jnp.int32, (NIR, ncols), 1)
    row = lax.broadcasted_iota(jnp.int32, (NIR, ncols), 0)
    r = jnp.where(col < MUL0, col,
        jnp.where(col < MUL0 + 3 * MUL1, MUL0 + (col - MUL0) // 3,
                  MUL0 + MUL1 + (col - MUL0 - 3 * MUL1) // 5))
    return (r == row).astype(jnp.float32)


def _node_body(xs_ref, xsph_ref, W1_ref, b1_ref, W2_ref, b2_ref,
               lng_ref, lnb_ref, o3g_ref, o3b_ref, w1e_ref, w2e_ref,
               ABM_ref, B_ref):
    xs = xs_ref[...]
    mu = jnp.mean(xs, axis=-1, keepdims=True)
    var = jnp.mean((xs - mu) * (xs - mu), axis=-1, keepdims=True)
    sin = (xs - mu) * lax.rsqrt(var + EPS) * lng_ref[...] + lnb_ref[...]
    h = jnp.dot(sin, W1_ref[...].T, preferred_element_type=jnp.float32) + b1_ref[...]
    h = h * jax.nn.sigmoid(h)
    so = jnp.dot(h, W2_ref[...].T, preferred_element_type=jnp.float32) + b2_ref[...]

    sph = xsph_ref[...]
    s = sph[:, :MUL0]
    mu2 = jnp.mean(s, axis=-1, keepdims=True)
    var2 = jnp.mean((s - mu2) * (s - mu2), axis=-1, keepdims=True)
    sn = (s - mu2) * lax.rsqrt(var2 + EPS) * o3g_ref[...] + o3b_ref[...]
    v1 = sph[:, MUL0:MUL0 + 3 * MUL1]
    inv1 = lax.rsqrt(jnp.sum(v1 * v1, axis=-1, keepdims=True) / MUL1 + EPS)
    v1n = v1 * inv1 * w1e_ref[...]
    v2 = sph[:, MUL0 + 3 * MUL1:]
    inv2 = lax.rsqrt(jnp.sum(v2 * v2, axis=-1, keepdims=True) / MUL2 + EPS)
    v2n = v2 * inv2 * w2e_ref[...]
    pad = jnp.zeros((sph.shape[0], DSPH_P - DSPH), jnp.float32)
    sph_in = jnp.concatenate([sn, v1n, v2n, pad], axis=-1)

    erep = _erep(DSPH_P)
    a = sph_in * jnp.dot(so[:, :NIR], erep, preferred_element_type=jnp.float32)
    ABM_ref[...] = jnp.concatenate([a, so[:, 2 * NIR:]], axis=-1)
    B_ref[...] = jnp.dot(so[:, NIR:2 * NIR], erep, preferred_element_type=jnp.float32)


def _edge_body(rbf_ref, fcut_ref, rsh_ref, Wr_ref, br_ref, FWX_ref):
    fw = (jnp.dot(rbf_ref[...], Wr_ref[...].T, preferred_element_type=jnp.float32)
          + br_ref[...]) * fcut_ref[...]
    erep = _erep(DSPH_P)
    fws = jnp.dot(fw[:, :NIR], erep, preferred_element_type=jnp.float32)
    pad = jnp.zeros((fw.shape[0], DSPH_P - DSPH), jnp.float32)
    rshp = jnp.concatenate([rsh_ref[...], pad], axis=-1)
    rbe = rshp * jnp.dot(fw[:, NIR:2 * NIR], erep,
                         preferred_element_type=jnp.float32)
    FWX_ref[...] = jnp.concatenate([fws, rbe, fw[:, 2 * NIR:]], axis=-1)


def _gather_body(ABM_h, B_h, FWX_h, src_h,
                 MS_h, M0_h, M1_h, M2_h, M3_h,
                 idx_all, ab_v, b_v, fwx_v,
                 oms_v, os0_v, os1_v, os2_v, os3_v, sem_in, sem_out):
    wid = lax.axis_index("s") * NCORES + lax.axis_index("c")
    base0 = wid * EPW
    zeros = jnp.zeros((LANES,), jnp.float32)
    pltpu.sync_copy(src_h.at[pl.ds(base0, EPW)], idx_all)

    obufs = (os0_v, os1_v, os2_v, os3_v)
    opairs = ((oms_v, MS_h), (os0_v, M0_h), (os1_v, M1_h),
              (os2_v, M2_h), (os3_v, M3_h))

    def chunk(j, carry):
        base = base0 + j * KA
        idx = idx_all.at[pl.ds(j * KA, KA)]
        d_ab = pltpu.async_copy(ABM_h.at[idx], ab_v, sem_in)
        d_b = pltpu.async_copy(B_h.at[idx], b_v, sem_in)
        d_fx = pltpu.async_copy(FWX_h.at[pl.ds(base, KA)], fwx_v, sem_in)

        @pl.when(j > 0)
        def _():
            for buf, hbm in opairs:
                pltpu.make_async_copy(buf, hbm.at[pl.ds(base, KA)],
                                      sem_out).wait()

        d_ab.wait()
        d_b.wait()
        d_fx.wait()

        @plsc.parallel_loop(0, KA, unroll=4)
        def row(r):
            for c in range(DSPH // LANES):
                t = c * LANES
                val = (ab_v[r, pl.ds(t, LANES)] * fwx_v[r, pl.ds(t, LANES)]
                       + b_v[r, pl.ds(t, LANES)]
                       * fwx_v[r, pl.ds(DSPH_P + t, LANES)])
                obufs[c // 8][r, pl.ds((c % 8) * LANES, LANES)] = val
            os3_v[r, pl.ds(96, LANES)] = zeros
            os3_v[r, pl.ds(112, LANES)] = zeros
            for c in range(ND // LANES):
                t = c * LANES
                oms_v[r, pl.ds(t, LANES)] = (
                    ab_v[r, pl.ds(DSPH_P + t, LANES)]
                    * fwx_v[r, pl.ds(2 * DSPH_P + t, LANES)])

        for buf, hbm in opairs:
            pltpu.async_copy(buf, hbm.at[pl.ds(base, KA)], sem_out)
        return carry

    lax.fori_loop(0, EPW // KA, chunk, 0)
    last = base0 + EPW - KA
    for buf, hbm in opairs:
        pltpu.make_async_copy(buf, hbm.at[pl.ds(last, KA)], sem_out).wait()


def _scatter_body(MS_h, M0_h, M1_h, M2_h, M3_h, dst_h, XS_h, XSPH_h,
                  OMS_h, OSPH_h,
                  acc_sh, idxa_v, idxb_v, rowsa_v, rowsb_v,
                  sema, semb, sems):
    core = lax.axis_index("c")
    sub = lax.axis_index("s")

    def do_chunk(msg_h, x_h, xcol, xw, out_h, ocol):
        pltpu.sync_copy(x_h.at[pl.ds(sub * RPT, RPT), pl.ds(xcol, xw)],
                        acc_sh.at[pl.ds(sub * RPT, RPT), pl.ds(0, xw)])

        @pl.when(sub == NSUB - 1)
        def _():
            pltpu.sync_copy(
                x_h.at[pl.ds(NSUB * RPT, N - NSUB * RPT), pl.ds(xcol, xw)],
                acc_sh.at[pl.ds(NSUB * RPT, N - NSUB * RPT), pl.ds(0, xw)])

        plsc.subcore_barrier()
        base0 = sub * EPT
        npairs = EPT // (2 * KB)

        def pair(p, carry):
            basea = base0 + p * 2 * KB
            baseb = basea + KB

            @pl.when(p > 0)
            def _():
                pltpu.make_async_copy(rowsa_v, acc_sh.at[idxa_v], sems).wait()
                pltpu.make_async_copy(rowsb_v, acc_sh.at[idxb_v], sems).wait()

            da1 = pltpu.async_copy(dst_h.at[pl.ds(basea, KB)], idxa_v, sema)
            da2 = pltpu.async_copy(msg_h.at[pl.ds(basea, KB)], rowsa_v, sema)
            db1 = pltpu.async_copy(dst_h.at[pl.ds(baseb, KB)], idxb_v, semb)
            db2 = pltpu.async_copy(msg_h.at[pl.ds(baseb, KB)], rowsb_v, semb)
            da1.wait()
            da2.wait()
            pltpu.async_copy(rowsa_v, acc_sh.at[idxa_v], sems, add=True)
            db1.wait()
            db2.wait()
            pltpu.async_copy(rowsb_v, acc_sh.at[idxb_v], sems, add=True)
            return carry

        lax.fori_loop(0, npairs, pair, 0)
        pltpu.make_async_copy(rowsa_v, acc_sh.at[idxa_v], sems).wait()
        pltpu.make_async_copy(rowsb_v, acc_sh.at[idxb_v], sems).wait()
        plsc.subcore_barrier()
        pltpu.sync_copy(acc_sh.at[pl.ds(sub * RPT, RPT)],
                        out_h.at[pl.ds(sub * RPT, RPT), pl.ds(ocol, CW)])

        @pl.when(sub == NSUB - 1)
        def _():
            pltpu.sync_copy(
                acc_sh.at[pl.ds(NSUB * RPT, N - NSUB * RPT)],
                out_h.at[pl.ds(NSUB * RPT, N - NSUB * RPT), pl.ds(ocol, CW)])

        plsc.subcore_barrier()

    @pl.when(core == 0)
    def _():
        do_chunk(MS_h, XS_h, 0, CW, OMS_h, 0)
        do_chunk(M0_h, XSPH_h, 0, CW, OSPH_h, 0)
        do_chunk(M1_h, XSPH_h, CW, CW, OSPH_h, CW)

    @pl.when(core == 1)
    def _():
        do_chunk(M2_h, XSPH_h, 2 * CW, CW, OSPH_h, 2 * CW)
        do_chunk(M3_h, XSPH_h, 3 * CW, CW, OSPH_h, 3 * CW)


def _f32(*shape):
    return jax.ShapeDtypeStruct(shape, jnp.float32)


def kernel(x_scalar, x_spherical, rbf, fcut, rsh, edge_index,
           W1, b1, W2, b2, Wr, br, ln_g, ln_b, o3_g, o3_b, o3_w1, o3_w2):
    src = edge_index[1]
    dst = edge_index[0]
    w1e = jnp.repeat(o3_w1, 3).reshape(1, -1)
    w2e = jnp.repeat(o3_w2, 5).reshape(1, -1)

    full = lambda s: pl.BlockSpec(s, lambda i: (0, 0))
    ABM, B = pl.pallas_call(
        _node_body,
        grid=(N // RB,),
        in_specs=[
            pl.BlockSpec((RB, ND), lambda i: (i, 0)),
            pl.BlockSpec((RB, DSPH), lambda i: (i, 0)),
            full((ND, ND)), full((1, ND)), full((HID, ND)), full((1, HID)),
            full((1, ND)), full((1, ND)), full((1, MUL0)), full((1, MUL0)),
            full((1, 3 * MUL1)), full((1, 5 * MUL2)),
        ],
        out_specs=[
            pl.BlockSpec((RB, DABM), lambda i: (i, 0)),
            pl.BlockSpec((RB, DSPH_P), lambda i: (i, 0)),
        ],
        out_shape=[_f32(N, DABM), _f32(N, DSPH_P)],
    )(x_scalar, x_spherical, W1, b1.reshape(1, -1), W2, b2.reshape(1, -1),
      ln_g.reshape(1, -1), ln_b.reshape(1, -1), o3_g.reshape(1, -1),
      o3_b.reshape(1, -1), w1e, w2e)

    FWX, = pl.pallas_call(
        _edge_body,
        grid=(E // EB,),
        in_specs=[
            pl.BlockSpec((EB, NBASIS), lambda i: (i, 0)),
            pl.BlockSpec((EB, 1), lambda i: (i, 0)),
            pl.BlockSpec((EB, DSPH), lambda i: (i, 0)),
            full((HID, NBASIS)), full((1, HID)),
        ],
        out_specs=[
            pl.BlockSpec((EB, 2 * DSPH_P + ND), lambda i: (i, 0)),
        ],
        out_shape=[_f32(E, 2 * DSPH_P + ND)],
    )(rbf, fcut, rsh, Wr, br.reshape(1, -1))

    mesh = plsc.VectorSubcoreMesh(core_axis_name="c", subcore_axis_name="s")

    MS, M0, M1, M2, M3 = pl.kernel(
        _gather_body,
        out_type=[_f32(E, CW)] * 5,
        mesh=mesh,
        scratch_types=[
            pltpu.VMEM((EPW,), jnp.int32),
            pltpu.VMEM((KA, DABM), jnp.float32),
            pltpu.VMEM((KA, DSPH_P), jnp.float32),
            pltpu.VMEM((KA, 2 * DSPH_P + ND), jnp.float32),
            pltpu.VMEM((KA, CW), jnp.float32),
            pltpu.VMEM((KA, CW), jnp.float32),
            pltpu.VMEM((KA, CW), jnp.float32),
            pltpu.VMEM((KA, CW), jnp.float32),
            pltpu.VMEM((KA, CW), jnp.float32),
            pltpu.SemaphoreType.DMA,
            pltpu.SemaphoreType.DMA,
        ],
    )(ABM, B, FWX, src)

    oms, osph = pl.kernel(
        _scatter_body,
        out_type=[_f32(N, CW), _f32(N, 4 * CW)],
        mesh=mesh,
        scratch_types=[
            pltpu.VMEM_SHARED((N, CW), jnp.float32),
            pltpu.VMEM((KB,), jnp.int32),
            pltpu.VMEM((KB,), jnp.int32),
            pltpu.VMEM((KB, CW), jnp.float32),
            pltpu.VMEM((KB, CW), jnp.float32),
            pltpu.SemaphoreType.DMA,
            pltpu.SemaphoreType.DMA,
            pltpu.SemaphoreType.DMA,
        ],
    )(MS, M0, M1, M2, M3, dst, x_scalar,
      jnp.pad(x_spherical, ((0, 0), (0, 4 * CW - DSPH))))

    return (oms, osph[:, :DSPH])
